# Initial kernel scaffold; baseline (speedup 1.0000x reference)
import jax
import jax.numpy as jnp
from jax.experimental import pallas as pl


def kernel(x, W_router, b_router, W_e1, b_e1, W_e2, b_e2, W_p1, b_p1, ln_g, ln_b, W_p2, b_p2):
    raise NotImplementedError("write your pallas kernel here")


# speedup vs baseline: 2.0014x; 2.0014x over previous
import functools

import jax
import jax.numpy as jnp
from jax.experimental import pallas as pl
from jax.experimental.pallas import tpu as pltpu

B = 4096
D = 768
E = 8
D_ADAPT = 256
D_PROJ = 1024

BLK = 1024


def _body(x_ref, Wr_ref, br_ref, We1_ref, be1_ref, We2_ref, be2_ref,
          Wp1_ref, bp1_ref, lng_ref, lnb_ref, Wp2_ref, bp2_ref, out_ref):
    f32 = jnp.float32
    x = x_ref[...]

    logits = jnp.dot(x, Wr_ref[...], preferred_element_type=f32) + br_ref[...]
    m = jnp.max(logits, axis=-1, keepdims=True)
    ex = jnp.exp(logits - m)
    probs = ex / jnp.sum(ex, axis=-1, keepdims=True)

    idx = jax.lax.broadcasted_iota(jnp.int32, (BLK, E), 1)
    top1 = jnp.max(probs, axis=-1, keepdims=True)
    i1 = jnp.min(jnp.where(probs == top1, idx, E), axis=-1, keepdims=True)
    probs2 = jnp.where(idx == i1, -jnp.inf, probs)
    top2 = jnp.max(probs2, axis=-1, keepdims=True)
    i2 = jnp.min(jnp.where(probs2 == top2, idx, E), axis=-1, keepdims=True)
    mask = (idx == i1) | (idx == i2)
    w = jnp.where(mask, probs, 0.0)
    w = w / (jnp.sum(w, axis=-1, keepdims=True) + 1e-9)

    acc = jnp.zeros((BLK, D), dtype=f32)
    for e in range(E):
        h = jnp.dot(x, We1_ref[e], preferred_element_type=f32) + be1_ref[e]
        h = jnp.maximum(h, 0.0)
        eo = jnp.dot(h, We2_ref[e], preferred_element_type=f32) + be2_ref[e]
        acc = acc + w[:, e:e + 1] * eo
    sw = jnp.sum(w, axis=-1, keepdims=True)
    fused = acc + sw * x

    p = jnp.dot(fused, Wp1_ref[...], preferred_element_type=f32) + bp1_ref[...]
    mu = jnp.mean(p, axis=-1, keepdims=True)
    var = jnp.mean((p - mu) ** 2, axis=-1, keepdims=True)
    p = (p - mu) / jnp.sqrt(var + 1e-5) * lng_ref[...] + lnb_ref[...]
    p = jnp.maximum(p, 0.0)
    out_ref[...] = jnp.dot(p, Wp2_ref[...], preferred_element_type=f32) + bp2_ref[...]


@jax.jit
def kernel(x, W_router, b_router, W_e1, b_e1, W_e2, b_e2,
           W_p1, b_p1, ln_g, ln_b, W_p2, b_p2):
    grid = (B // BLK,)
    fixed = lambda shape: pl.BlockSpec(shape, lambda i: (0,) * len(shape))
    return pl.pallas_call(
        _body,
        grid=grid,
        in_specs=[
            pl.BlockSpec((BLK, D), lambda i: (i, 0)),
            fixed((D, E)),
            fixed((1, E)),
            fixed((E, D, D_ADAPT)),
            fixed((E, 1, D_ADAPT)),
            fixed((E, D_ADAPT, D)),
            fixed((E, 1, D)),
            fixed((D, D_PROJ)),
            fixed((1, D_PROJ)),
            fixed((1, D_PROJ)),
            fixed((1, D_PROJ)),
            fixed((D_PROJ, D)),
            fixed((1, D)),
        ],
        out_specs=pl.BlockSpec((BLK, D), lambda i: (i, 0)),
        out_shape=jax.ShapeDtypeStruct((B, D), jnp.float32),
        compiler_params=pltpu.CompilerParams(
            dimension_semantics=("arbitrary",),
        ),
    )(x, W_router, b_router.reshape(1, E),
      W_e1, b_e1.reshape(E, 1, D_ADAPT), W_e2, b_e2.reshape(E, 1, D),
      W_p1, b_p1.reshape(1, D_PROJ), ln_g.reshape(1, D_PROJ),
      ln_b.reshape(1, D_PROJ), W_p2, b_p2.reshape(1, D))
